# baseline (speedup 1.0000x reference)
import functools
import jax
import jax.numpy as jnp
from jax import lax
from jax.experimental import pallas as pl
from jax.experimental.pallas import tpu as pltpu
from jax.experimental.pallas import tpu_sc as plsc

H = 128
NUM_F = 8
CAT_F = 6
BIN_F = 6
TOKENS = NUM_F + CAT_F + BIN_F
GATHER_F = CAT_F + BIN_F
NBUF = 8
AHEAD = 4
L = 16
NJ = H // L
VOCABS = (151, 101, 21, 5, 4, 15)


def kernel(numeric, categorical, binary, W_num, b_num, bin_emb,
           cat_emb_0, cat_emb_1, cat_emb_2, cat_emb_3, cat_emb_4, cat_emb_5):
    B = numeric.shape[0]
    info = plsc.get_sparse_core_info()
    NC, NS = info.num_cores, info.num_subcores
    NW = NC * NS
    rows_w = B // NW

    num_rp = jnp.pad(numeric, ((0, 0), (0, L - NUM_F)))
    num_rp = num_rp.reshape(NW, rows_w * L)

    tabs_all = jnp.concatenate(
        [cat_emb_0, cat_emb_1, cat_emb_2, cat_emb_3, cat_emb_4, cat_emb_5,
         bin_emb.reshape(2 * BIN_F, H)], axis=0)
    cat_offs = [0]
    for v in VOCABS:
        cat_offs.append(cat_offs[-1] + v)
    coffs = jnp.asarray(cat_offs[:CAT_F], jnp.int32)[None, :]
    boffs = (cat_offs[CAT_F]
             + 2 * jnp.arange(BIN_F, dtype=jnp.int32))[None, :]
    idx_rm = jnp.concatenate(
        [categorical.astype(jnp.int32) + coffs,
         binary.astype(jnp.int32) + boffs,
         jnp.zeros((B, L - GATHER_F), jnp.int32)], axis=1)
    idx_rm = idx_rm.reshape(NW, rows_w * L)

    mesh = plsc.VectorSubcoreMesh(core_axis_name="c", subcore_axis_name="s")

    @functools.partial(
        pl.kernel, mesh=mesh,
        out_type=jax.ShapeDtypeStruct((B, TOKENS, H), jnp.float32),
        scratch_types=[
            pltpu.VMEM((NBUF, TOKENS, H), jnp.float32),
            pltpu.VMEM((rows_w * L,), jnp.float32),
            pltpu.VMEM((rows_w * L,), jnp.int32),
            pltpu.VMEM((NUM_F, H), jnp.float32),
            pltpu.VMEM((NUM_F, H), jnp.float32),
            pltpu.SemaphoreType.DMA,
            pltpu.SemaphoreType.DMA,
        ],
    )
    def sck(num_hbm, idx_hbm, wn_hbm, bn_hbm, tab_hbm,
            out_hbm, staging, num_v, idx_v, w_v, b_v, gsem, wsem):
        wid = lax.axis_index("s") * NC + lax.axis_index("c")
        base = wid * rows_w
        pltpu.sync_copy(num_hbm.at[wid], num_v)
        pltpu.sync_copy(idx_hbm.at[wid], idx_v)
        pltpu.sync_copy(wn_hbm, w_v)
        pltpu.sync_copy(bn_hbm, b_v)

        def issue_gather(rr):
            pltpu.async_copy(
                tab_hbm.at[idx_v.at[pl.ds(rr * L, GATHER_F)]],
                staging.at[rr % NBUF, pl.ds(NUM_F, GATHER_F), :], gsem)

        def wait_gather():
            pltpu.make_async_copy(
                tab_hbm.at[idx_v.at[pl.ds(0, GATHER_F)]],
                staging.at[0, pl.ds(NUM_F, GATHER_F), :], gsem).wait()

        def issue_write(rr):
            pltpu.async_copy(
                staging.at[rr % NBUF], out_hbm.at[base + rr], wsem)

        def wait_write():
            pltpu.make_async_copy(
                staging.at[0], out_hbm.at[base], wsem).wait()

        for rr in range(AHEAD):
            issue_gather(rr)

        wvecs = [[w_v[t, pl.ds(L * j, L)] for j in range(NJ)]
                 for t in range(NUM_F)]
        bvecs = [[b_v[t, pl.ds(L * j, L)] for j in range(NJ)]
                 for t in range(NUM_F)]

        def do_row(r, _):
            smod = r % NBUF

            @pl.when(r >= AHEAD)
            def _drain():
                wait_write()

            @pl.when(r + AHEAD < rows_w)
            def _prefetch():
                issue_gather(r + AHEAD)

            xv = num_v[pl.ds(r * L, L)]
            for t in range(NUM_F):
                xsp = jnp.full((L,), xv[t], jnp.float32)
                for j in range(NJ):
                    staging[smod, t, pl.ds(L * j, L)] = (
                        xsp * wvecs[t][j] + bvecs[t][j])

            wait_gather()
            issue_write(r)
            return 0

        lax.fori_loop(0, rows_w, do_row, 0)
        for _ in range(AHEAD):
            wait_write()

    return sck(num_rp, idx_rm, W_num, b_num, tabs_all)


# speedup vs baseline: 1.6560x; 1.6560x over previous
import functools
import jax
import jax.numpy as jnp
from jax import lax
from jax.experimental import pallas as pl
from jax.experimental.pallas import tpu as pltpu
from jax.experimental.pallas import tpu_sc as plsc

H = 128
NUM_F = 8
CAT_F = 6
BIN_F = 6
TOKENS = NUM_F + CAT_F + BIN_F
CH = 8
L = 16
NJ = H // L
VOCABS = (151, 101, 21, 5, 4, 15)


def kernel(numeric, categorical, binary, W_num, b_num, bin_emb,
           cat_emb_0, cat_emb_1, cat_emb_2, cat_emb_3, cat_emb_4, cat_emb_5):
    B = numeric.shape[0]
    info = plsc.get_sparse_core_info()
    NC, NS = info.num_cores, info.num_subcores
    NW = NC * NS
    rows_w = B // NW
    nch = rows_w // CH

    num_cp = (numeric.reshape(NW, nch, CH, NUM_F)
              .transpose(0, 1, 3, 2))
    num_cp = jnp.pad(num_cp, ((0, 0), (0, 0), (0, 0), (0, L - CH)))
    num_cp = num_cp.reshape(NW, nch * NUM_F * L)

    boffs = (jnp.arange(BIN_F, dtype=jnp.int32) * 2 * H)[None, :]
    idx16 = jnp.concatenate(
        [categorical.astype(jnp.int32) * H,
         binary.astype(jnp.int32) * H + boffs,
         jnp.zeros((B, L - CAT_F - BIN_F), jnp.int32)], axis=1)
    idx_cp = idx16.reshape(NW, rows_w * L)

    cat_tables = [cat_emb_0.reshape(-1), cat_emb_1.reshape(-1),
                  cat_emb_2.reshape(-1), cat_emb_3.reshape(-1),
                  cat_emb_4.reshape(-1), cat_emb_5.reshape(-1)]
    be_flat = bin_emb.reshape(BIN_F * 2 * H)
    mesh = plsc.VectorSubcoreMesh(core_axis_name="c", subcore_axis_name="s")

    @functools.partial(
        pl.kernel, mesh=mesh,
        out_type=jax.ShapeDtypeStruct((B, TOKENS, H), jnp.float32),
        scratch_types=[
            pltpu.VMEM((2, CH, TOKENS, H), jnp.float32),
            pltpu.VMEM((nch * NUM_F * L,), jnp.float32),
            pltpu.VMEM((rows_w * L,), jnp.int32),
            pltpu.VMEM((NUM_F, H), jnp.float32),
            pltpu.VMEM((NUM_F, H), jnp.float32),
            pltpu.VMEM((BIN_F * 2 * H,), jnp.float32),
        ] + [pltpu.VMEM((v * H,), jnp.float32) for v in VOCABS]
        + [pltpu.SemaphoreType.DMA],
    )
    def sck(num_hbm, idx_hbm, wn_hbm, bn_hbm, be_hbm,
            ct0_hbm, ct1_hbm, ct2_hbm, ct3_hbm, ct4_hbm, ct5_hbm,
            out_hbm, staging, num_v, idx_v, w_v, b_v, be_v,
            ct0, ct1, ct2, ct3, ct4, ct5, sem):
        wid = lax.axis_index("s") * NC + lax.axis_index("c")
        base = wid * rows_w
        ctabs = [ct0, ct1, ct2, ct3, ct4, ct5]
        ct_hbms = [ct0_hbm, ct1_hbm, ct2_hbm, ct3_hbm, ct4_hbm, ct5_hbm]
        pltpu.sync_copy(num_hbm.at[wid], num_v)
        pltpu.sync_copy(idx_hbm.at[wid], idx_v)
        pltpu.sync_copy(wn_hbm, w_v)
        pltpu.sync_copy(bn_hbm, b_v)
        pltpu.sync_copy(be_hbm, be_v)
        for i in range(CAT_F):
            pltpu.sync_copy(ct_hbms[i], ctabs[i])

        def do_chunk(c, _):
            cmod = c % 2

            @pl.when(c >= 2)
            def _drain():
                pltpu.make_async_copy(
                    staging.at[0], out_hbm.at[pl.ds(base, CH)], sem).wait()

            wvecs = [[w_v[t, pl.ds(L * j, L)] for j in range(NJ)]
                     for t in range(NUM_F)]
            bvecs = [[b_v[t, pl.ds(L * j, L)] for j in range(NJ)]
                     for t in range(NUM_F)]
            xrows = [num_v[pl.ds(c * (NUM_F * L) + t * L, L)]
                     for t in range(NUM_F)]

            def fill(smod):
                for rl in range(CH):
                    iv = idx_v[pl.ds(c * (CH * L) + rl * L, L)]
                    for t in range(NUM_F):
                        xsp = jnp.full((L,), xrows[t][rl], jnp.float32)
                        for j in range(NJ):
                            staging[smod, rl, t, pl.ds(L * j, L)] = (
                                xsp * wvecs[t][j] + bvecs[t][j])
                    for i in range(CAT_F):
                        ibase = iv[i]
                        for j in range(NJ):
                            staging[smod, rl, NUM_F + i, pl.ds(L * j, L)] = (
                                ctabs[i][pl.ds(ibase + L * j, L)])
                    for i in range(BIN_F):
                        bbase = iv[CAT_F + i]
                        for j in range(NJ):
                            staging[smod, rl, NUM_F + CAT_F + i,
                                    pl.ds(L * j, L)] = (
                                be_v[pl.ds(bbase + L * j, L)])

            @pl.when(cmod == 0)
            def _fill0():
                fill(0)

            @pl.when(cmod == 1)
            def _fill1():
                fill(1)

            pltpu.async_copy(
                staging.at[cmod], out_hbm.at[pl.ds(base + c * CH, CH)], sem)
            return 0

        lax.fori_loop(0, nch, do_chunk, 0)
        for _ in range(2):
            pltpu.make_async_copy(
                staging.at[0], out_hbm.at[pl.ds(base, CH)], sem).wait()

    return sck(num_cp, idx_cp, W_num, b_num, be_flat, *cat_tables)


# speedup vs baseline: 2.1373x; 1.2906x over previous
import functools
import jax
import jax.numpy as jnp
from jax import lax
from jax.experimental import pallas as pl
from jax.experimental.pallas import tpu as pltpu
from jax.experimental.pallas import tpu_sc as plsc

H = 128
NUM_F = 8
CAT_F = 6
BIN_F = 6
TOKENS = NUM_F + CAT_F + BIN_F
CH = 8
L = 16
NJ = H // L
VOCABS = (151, 101, 21, 5, 4, 15)


def kernel(numeric, categorical, binary, W_num, b_num, bin_emb,
           cat_emb_0, cat_emb_1, cat_emb_2, cat_emb_3, cat_emb_4, cat_emb_5):
    B = numeric.shape[0]
    info = plsc.get_sparse_core_info()
    NC, NS = info.num_cores, info.num_subcores
    NW = NC * NS
    rows_w = B // NW
    nch = rows_w // CH

    num_cp = jnp.pad(numeric, ((0, 0), (0, L - NUM_F)))
    num_cp = num_cp.reshape(NW, rows_w * L)

    boffs = (jnp.arange(BIN_F, dtype=jnp.int32) * 2 * H)[None, :]
    idx16 = jnp.concatenate(
        [categorical.astype(jnp.int32) * H,
         binary.astype(jnp.int32) * H + boffs,
         jnp.zeros((B, L - CAT_F - BIN_F), jnp.int32)], axis=1)
    idx_cp = idx16.reshape(NW, rows_w * L)

    cat_tables = [cat_emb_0.reshape(-1), cat_emb_1.reshape(-1),
                  cat_emb_2.reshape(-1), cat_emb_3.reshape(-1),
                  cat_emb_4.reshape(-1), cat_emb_5.reshape(-1)]
    be_flat = bin_emb.reshape(BIN_F * 2 * H)
    mesh = plsc.VectorSubcoreMesh(core_axis_name="c", subcore_axis_name="s")

    @functools.partial(
        pl.kernel, mesh=mesh,
        out_type=jax.ShapeDtypeStruct((B, TOKENS, H), jnp.float32),
        scratch_types=[
            pltpu.VMEM((2, CH, TOKENS, H), jnp.float32),
            pltpu.VMEM((rows_w * L,), jnp.float32),
            pltpu.VMEM((rows_w * L,), jnp.int32),
            pltpu.VMEM((NUM_F, H), jnp.float32),
            pltpu.VMEM((NUM_F, H), jnp.float32),
            pltpu.VMEM((BIN_F * 2 * H,), jnp.float32),
        ] + [pltpu.VMEM((v * H,), jnp.float32) for v in VOCABS]
        + [pltpu.SemaphoreType.DMA],
    )
    def sck(num_hbm, idx_hbm, wn_hbm, bn_hbm, be_hbm,
            ct0_hbm, ct1_hbm, ct2_hbm, ct3_hbm, ct4_hbm, ct5_hbm,
            out_hbm, staging, num_v, idx_v, w_v, b_v, be_v,
            ct0, ct1, ct2, ct3, ct4, ct5, sem):
        wid = lax.axis_index("s") * NC + lax.axis_index("c")
        base = wid * rows_w
        ctabs = [ct0, ct1, ct2, ct3, ct4, ct5]
        ct_hbms = [ct0_hbm, ct1_hbm, ct2_hbm, ct3_hbm, ct4_hbm, ct5_hbm]
        pltpu.sync_copy(num_hbm.at[wid], num_v)
        pltpu.sync_copy(idx_hbm.at[wid], idx_v)
        pltpu.sync_copy(wn_hbm, w_v)
        pltpu.sync_copy(bn_hbm, b_v)
        pltpu.sync_copy(be_hbm, be_v)
        for i in range(CAT_F):
            pltpu.sync_copy(ct_hbms[i], ctabs[i])

        def do_chunk(c, _):
            cmod = c % 2
            sref = staging.at[cmod]

            @pl.when(c >= 2)
            def _drain():
                pltpu.make_async_copy(
                    staging.at[0], out_hbm.at[pl.ds(base, CH)], sem).wait()

            wvecs = [[w_v[t, pl.ds(L * j, L)] for j in range(NJ)]
                     for t in range(NUM_F)]
            bvecs = [[b_v[t, pl.ds(L * j, L)] for j in range(NJ)]
                     for t in range(NUM_F)]

            @plsc.parallel_loop(0, CH, step=1)
            def _rows(rl):
                iv = idx_v[pl.ds(c * (CH * L) + rl * L, L)]
                xv = num_v[pl.ds(c * (CH * L) + rl * L, L)]
                for t in range(NUM_F):
                    xsp = jnp.full((L,), xv[t], jnp.float32)
                    for j in range(NJ):
                        sref[rl, t, pl.ds(L * j, L)] = (
                            xsp * wvecs[t][j] + bvecs[t][j])
                for i in range(CAT_F):
                    ibase = iv[i]
                    for j in range(NJ):
                        sref[rl, NUM_F + i, pl.ds(L * j, L)] = (
                            ctabs[i][pl.ds(ibase + L * j, L)])
                for i in range(BIN_F):
                    bbase = iv[CAT_F + i]
                    for j in range(NJ):
                        sref[rl, NUM_F + CAT_F + i, pl.ds(L * j, L)] = (
                            be_v[pl.ds(bbase + L * j, L)])

            pltpu.async_copy(
                staging.at[cmod], out_hbm.at[pl.ds(base + c * CH, CH)], sem)
            return 0

        lax.fori_loop(0, nch, do_chunk, 0)
        for _ in range(2):
            pltpu.make_async_copy(
                staging.at[0], out_hbm.at[pl.ds(base, CH)], sem).wait()

    return sck(num_cp, idx_cp, W_num, b_num, be_flat, *cat_tables)


# speedup vs baseline: 2.4429x; 1.1430x over previous
import functools
import jax
import jax.numpy as jnp
from jax import lax
from jax.experimental import pallas as pl
from jax.experimental.pallas import tpu as pltpu
from jax.experimental.pallas import tpu_sc as plsc

H = 128
NUM_F = 8
CAT_F = 6
BIN_F = 6
TOKENS = NUM_F + CAT_F + BIN_F
CH = 8
L = 16
NJ = H // L
VOCABS = (151, 101, 21, 5, 4, 15)


def kernel(numeric, categorical, binary, W_num, b_num, bin_emb,
           cat_emb_0, cat_emb_1, cat_emb_2, cat_emb_3, cat_emb_4, cat_emb_5):
    B = numeric.shape[0]
    info = plsc.get_sparse_core_info()
    NC, NS = info.num_cores, info.num_subcores
    NW = NC * NS
    rows_w = B // NW
    nch = rows_w // CH

    num_cp = jnp.pad(numeric, ((0, 0), (0, L - NUM_F)))
    num_cp = num_cp.reshape(NW, rows_w * L)

    boffs = (jnp.arange(BIN_F, dtype=jnp.int32) * 2 * H)[None, :]
    idx16 = jnp.concatenate(
        [categorical.astype(jnp.int32) * H,
         binary.astype(jnp.int32) * H + boffs,
         jnp.zeros((B, L - CAT_F - BIN_F), jnp.int32)], axis=1)
    idx_cp = idx16.reshape(NW, rows_w * L)

    cat_tables = [cat_emb_0.reshape(-1), cat_emb_1.reshape(-1),
                  cat_emb_2.reshape(-1), cat_emb_3.reshape(-1),
                  cat_emb_4.reshape(-1), cat_emb_5.reshape(-1)]
    be_flat = bin_emb.reshape(BIN_F * 2 * H)
    mesh = plsc.VectorSubcoreMesh(core_axis_name="c", subcore_axis_name="s")

    @functools.partial(
        pl.kernel, mesh=mesh,
        out_type=jax.ShapeDtypeStruct((B, TOKENS, H), jnp.float32),
        scratch_types=[
            pltpu.VMEM((2, CH, TOKENS, H), jnp.float32),
            pltpu.VMEM((rows_w * L,), jnp.float32),
            pltpu.VMEM((rows_w * L,), jnp.int32),
            pltpu.VMEM((NUM_F, H), jnp.float32),
            pltpu.VMEM((NUM_F, H), jnp.float32),
            pltpu.VMEM((BIN_F * 2 * H,), jnp.float32),
        ] + [pltpu.VMEM((v * H,), jnp.float32) for v in VOCABS]
        + [pltpu.SemaphoreType.DMA],
    )
    def sck(num_hbm, idx_hbm, wn_hbm, bn_hbm, be_hbm,
            ct0_hbm, ct1_hbm, ct2_hbm, ct3_hbm, ct4_hbm, ct5_hbm,
            out_hbm, staging, num_v, idx_v, w_v, b_v, be_v,
            ct0, ct1, ct2, ct3, ct4, ct5, sem):
        wid = lax.axis_index("s") * NC + lax.axis_index("c")
        base = wid * rows_w
        ctabs = [ct0, ct1, ct2, ct3, ct4, ct5]
        ct_hbms = [ct0_hbm, ct1_hbm, ct2_hbm, ct3_hbm, ct4_hbm, ct5_hbm]
        pltpu.sync_copy(num_hbm.at[wid], num_v)
        pltpu.sync_copy(idx_hbm.at[wid], idx_v)
        pltpu.sync_copy(wn_hbm, w_v)
        pltpu.sync_copy(bn_hbm, b_v)
        pltpu.sync_copy(be_hbm, be_v)
        for i in range(CAT_F):
            pltpu.sync_copy(ct_hbms[i], ctabs[i])

        def do_chunk(c, _):
            cmod = c % 2
            sref = staging.at[cmod]

            @pl.when(c >= 2)
            def _drain():
                pltpu.make_async_copy(
                    staging.at[0], out_hbm.at[pl.ds(base, CH)], sem).wait()

            wvecs = [[w_v[t, pl.ds(L * j, L)] for j in range(NJ)]
                     for t in range(NUM_F)]
            bvecs = [[b_v[t, pl.ds(L * j, L)] for j in range(NJ)]
                     for t in range(NUM_F)]

            @plsc.parallel_loop(0, CH, step=1, unroll=4)
            def _rows(rl):
                iv = idx_v[pl.ds(c * (CH * L) + rl * L, L)]
                xv = num_v[pl.ds(c * (CH * L) + rl * L, L)]
                for t in range(NUM_F):
                    xsp = jnp.full((L,), xv[t], jnp.float32)
                    for j in range(NJ):
                        sref[rl, t, pl.ds(L * j, L)] = (
                            xsp * wvecs[t][j] + bvecs[t][j])
                for i in range(CAT_F):
                    ibase = iv[i]
                    for j in range(NJ):
                        sref[rl, NUM_F + i, pl.ds(L * j, L)] = (
                            ctabs[i][pl.ds(ibase + L * j, L)])
                for i in range(BIN_F):
                    bbase = iv[CAT_F + i]
                    for j in range(NJ):
                        sref[rl, NUM_F + CAT_F + i, pl.ds(L * j, L)] = (
                            be_v[pl.ds(bbase + L * j, L)])

            pltpu.async_copy(
                staging.at[cmod], out_hbm.at[pl.ds(base + c * CH, CH)], sem)
            return 0

        lax.fori_loop(0, nch, do_chunk, 0)
        for _ in range(2):
            pltpu.make_async_copy(
                staging.at[0], out_hbm.at[pl.ds(base, CH)], sem).wait()

    return sck(num_cp, idx_cp, W_num, b_num, be_flat, *cat_tables)
